# baseline (speedup 1.0000x reference)
import functools

import jax
import jax.numpy as jnp
from jax import lax
from jax.experimental import pallas as pl
from jax.experimental.pallas import tpu as pltpu
from jax.experimental.pallas import tpu_sc as plsc

_H = 32
_W = 32
_D = 768
_NC = 2
_NS = 16
_NW = _NC * _NS
_L = 16
_WT = _W // 8
_DT = _D // 128

_mesh = plsc.VectorSubcoreMesh(core_axis_name="c", subcore_axis_name="s")


@functools.partial(
    pl.kernel,
    mesh=_mesh,
    out_type=jax.ShapeDtypeStruct((_H, _WT, _DT, 8, 128), jnp.float32),
    compiler_params=pltpu.CompilerParams(
        use_tc_tiling_on_sc=False,
        needs_layout_passes=False,
        skip_device_barrier=True,
    ),
    scratch_types=[
        pltpu.VMEM((_WT, _DT, 8, 128), jnp.float32),
        pltpu.VMEM((_DT, 128), jnp.float32),
        pltpu.VMEM_SHARED((_WT, _DT, 8, 128), jnp.float32),
        pltpu.SemaphoreType.DMA,
        [pltpu.SemaphoreType.DMA] * _WT,
        pltpu.SemaphoreType.DMA,
    ],
)
def _pos2d(row_hbm, col_hbm, out_hbm, buf_v, row_v, col_sh, rsem, csems, osem):
    s = lax.axis_index("s")
    h = s * _NC + lax.axis_index("c")
    ht = h // 8
    hs = h % 8

    rcopy = pltpu.make_async_copy(row_hbm.at[ht, :, hs], row_v, rsem)
    rcopy.start()

    @pl.when(s < _WT)
    def _stage():
        pltpu.sync_copy(col_hbm.at[s], col_sh.at[s])

    plsc.subcore_barrier()
    rcopy.wait()

    ccopies = []
    for wt in range(_WT):
        c = pltpu.make_async_copy(col_sh.at[wt], buf_v.at[wt], csems[wt])
        c.start()
        ccopies.append(c)

    ocopies = []
    for wt in range(_WT):
        ccopies[wt].wait()

        @plsc.parallel_loop(0, _DT)
        def _add_rows(dt, wt=wt):
            rvec = [row_v[dt, pl.ds(_L * k, _L)] for k in range(8)]
            for ws in range(8):
                for k in range(8):
                    plsc.addupdate(
                        buf_v.at[wt, dt, ws, pl.ds(_L * k, _L)], rvec[k]
                    )

        o = pltpu.make_async_copy(buf_v.at[wt], out_hbm.at[h, wt], osem)
        o.start()
        ocopies.append(o)
    for o in ocopies:
        o.wait()


def kernel(row_embed, col_embed):
    row4 = row_embed.reshape(_WT, 8, _DT, 128).transpose(0, 2, 1, 3)
    col4 = col_embed.reshape(_WT, 8, _DT, 128).transpose(0, 2, 1, 3)
    out5 = _pos2d(row4, col4)
    p = out5.transpose(0, 1, 3, 2, 4).reshape(_H, _W, _D)
    return jnp.transpose(p, (2, 0, 1))


# speedup vs baseline: 1.0016x; 1.0016x over previous
import functools

import jax
import jax.numpy as jnp
from jax import lax
from jax.experimental import pallas as pl
from jax.experimental.pallas import tpu as pltpu
from jax.experimental.pallas import tpu_sc as plsc

_H = 32
_W = 32
_D = 768
_NC = 2
_NS = 16
_NW = _NC * _NS
_L = 16
_WT = _W // 8
_DT = _D // 128

_mesh = plsc.VectorSubcoreMesh(core_axis_name="c", subcore_axis_name="s")


@functools.partial(
    pl.kernel,
    mesh=_mesh,
    out_type=jax.ShapeDtypeStruct((_H, _WT, _DT, 8, 128), jnp.float32),
    compiler_params=pltpu.CompilerParams(
        use_tc_tiling_on_sc=False, needs_layout_passes=False
    ),
    scratch_types=[
        pltpu.VMEM((_WT, _DT, 8, 128), jnp.float32),
        pltpu.VMEM((_DT, 128), jnp.float32),
        pltpu.VMEM_SHARED((_WT, _DT, 8, 128), jnp.float32),
        pltpu.SemaphoreType.DMA,
        [pltpu.SemaphoreType.DMA] * _WT,
        pltpu.SemaphoreType.DMA,
    ],
)
def _pos2d(row_hbm, col_hbm, out_hbm, buf_v, row_v, col_sh, rsem, csems, osem):
    s = lax.axis_index("s")
    h = s * _NC + lax.axis_index("c")
    ht = h // 8
    hs = h % 8

    rcopy = pltpu.make_async_copy(row_hbm.at[ht, :, hs], row_v, rsem)
    rcopy.start()

    @pl.when(s < _WT)
    def _stage():
        pltpu.sync_copy(col_hbm.at[s], col_sh.at[s])

    plsc.subcore_barrier()
    rcopy.wait()

    wts = [(s + i) % _WT for i in range(_WT)]
    ccopies = []
    for i in range(_WT):
        c = pltpu.make_async_copy(
            col_sh.at[wts[i]], buf_v.at[wts[i]], csems[i]
        )
        c.start()
        ccopies.append(c)

    ocopies = []
    for i in range(_WT):
        wt = wts[i]
        ccopies[i].wait()

        @plsc.parallel_loop(0, _DT)
        def _add_rows(dt, wt=wt):
            rvec = [row_v[dt, pl.ds(_L * k, _L)] for k in range(8)]
            for ws in range(8):
                for k in range(8):
                    plsc.addupdate(
                        buf_v.at[wt, dt, ws, pl.ds(_L * k, _L)], rvec[k]
                    )

        o = pltpu.make_async_copy(buf_v.at[wt], out_hbm.at[h, wt], osem)
        o.start()
        ocopies.append(o)
    for o in ocopies:
        o.wait()


def kernel(row_embed, col_embed):
    row4 = row_embed.reshape(_WT, 8, _DT, 128).transpose(0, 2, 1, 3)
    col4 = col_embed.reshape(_WT, 8, _DT, 128).transpose(0, 2, 1, 3)
    out5 = _pos2d(row4, col4)
    p = out5.transpose(0, 1, 3, 2, 4).reshape(_H, _W, _D)
    return jnp.transpose(p, (2, 0, 1))
